# Initial kernel scaffold; baseline (speedup 1.0000x reference)
import jax
import jax.numpy as jnp
from jax.experimental import pallas as pl


def kernel(x, edge_index, W, b):
    raise NotImplementedError("write your pallas kernel here")


# speedup vs baseline: 3.8552x; 3.8552x over previous
import functools

import jax
import jax.numpy as jnp
from jax import lax
from jax.experimental import pallas as pl
from jax.experimental.pallas import tpu as pltpu
from jax.experimental.pallas import tpu_sc as plsc

N_NODES = 10000
N_EDGES = 320000
DIM = 128
DAUG = 144
NPAD = 10240
CHUNK = 128
NCHUNKS = 79
EDGES_PER_TILE = CHUNK * NCHUNKS
EPAD = EDGES_PER_TILE * 32
ROWS_PER_TILE = NPAD // 16
BM = 1000

_mesh = plsc.VectorSubcoreMesh(core_axis_name="c", subcore_axis_name="s")


@functools.partial(
    pl.kernel,
    out_type=jax.ShapeDtypeStruct((2 * NPAD, DAUG), jnp.float32),
    mesh=_mesh,
    compiler_params=pltpu.CompilerParams(use_tc_tiling_on_sc=False),
    scratch_types=[
        pltpu.VMEM_SHARED((NPAD, DAUG), jnp.float32),
        pltpu.VMEM((CHUNK,), jnp.int32),
        pltpu.VMEM((CHUNK,), jnp.int32),
        pltpu.VMEM((CHUNK, DAUG), jnp.float32),
        pltpu.VMEM((CHUNK, DAUG), jnp.float32),
        pltpu.SemaphoreType.DMA,
    ],
)
def _segsum_sc(xaug, srcp, dstp, out, acc, src_v, dst_v, rows_v, zbuf, sem):
    c = lax.axis_index("c")
    s = lax.axis_index("s")
    wid = s * 2 + c
    my_base = s * ROWS_PER_TILE

    zeros16 = jnp.zeros((16,), jnp.float32)

    def zrow(r, carry):
        for cc in range(DAUG // 16):
            zbuf[r, pl.ds(cc * 16, 16)] = zeros16
        return carry

    lax.fori_loop(0, CHUNK, zrow, 0)

    def zacc(j, carry):
        pltpu.sync_copy(zbuf, acc.at[pl.ds(my_base + j * CHUNK, CHUNK)])
        return carry

    lax.fori_loop(0, ROWS_PER_TILE // CHUNK, zacc, 0)
    plsc.subcore_barrier()

    ebase = wid * EDGES_PER_TILE

    def body(i, carry):
        off = ebase + i * CHUNK
        pltpu.sync_copy(srcp.at[pl.ds(off, CHUNK)], src_v)
        pltpu.sync_copy(dstp.at[pl.ds(off, CHUNK)], dst_v)
        pltpu.async_copy(xaug.at[src_v], rows_v, sem).wait()
        pltpu.sync_copy(rows_v, acc.at[dst_v], add=True)
        return carry

    lax.fori_loop(0, NCHUNKS, body, 0)
    plsc.subcore_barrier()

    pltpu.sync_copy(
        acc.at[pl.ds(my_base, ROWS_PER_TILE)],
        out.at[pl.ds(c * NPAD + my_base, ROWS_PER_TILE)],
    )


def _combine_body(p_ref, x_ref, w_ref, b_ref, f_ref, ssq_ref):
    p = p_ref[0] + p_ref[1]
    hsum = p[:, :DIM]
    deg = jnp.sum(p[:, DIM:], axis=1, keepdims=True)
    deg = jnp.maximum(deg, 1.0)
    h = hsum / deg
    w13 = w_ref[:DIM, :] + w_ref[2 * DIM:, :]
    w2 = w_ref[DIM:2 * DIM, :]
    z = jnp.dot(x_ref[...], w13, preferred_element_type=jnp.float32)
    z = z + jnp.dot(h, w2, preferred_element_type=jnp.float32)
    f = jnp.tanh(z + b_ref[...])
    f_ref[...] = f

    blk = jnp.sum(f * f)[None, None]

    @pl.when(pl.program_id(0) == 0)
    def _():
        ssq_ref[...] = blk

    @pl.when(pl.program_id(0) > 0)
    def _():
        ssq_ref[...] = ssq_ref[...] + blk


def _scale_body(f_ref, ssq_ref, o_ref):
    o_ref[...] = f_ref[...] * lax.rsqrt(ssq_ref[...])


def kernel(x, edge_index, W, b):
    x = x.astype(jnp.float32)
    src = edge_index[0].astype(jnp.int32)
    dst = edge_index[1].astype(jnp.int32)
    pad = EPAD - N_EDGES
    srcp = jnp.concatenate([src, jnp.zeros((pad,), jnp.int32)])
    dstp = jnp.concatenate([dst, jnp.full((pad,), N_NODES, jnp.int32)])
    xaug = jnp.concatenate(
        [x, jnp.ones((N_NODES, 1), jnp.float32),
         jnp.zeros((N_NODES, DAUG - DIM - 1), jnp.float32)], axis=1)

    partials = _segsum_sc(xaug, srcp, dstp).reshape(2, NPAD, DAUG)

    grid = N_NODES // BM
    f, ssq = pl.pallas_call(
        _combine_body,
        grid=(grid,),
        in_specs=[
            pl.BlockSpec((2, BM, DAUG), lambda i: (0, i, 0)),
            pl.BlockSpec((BM, DIM), lambda i: (i, 0)),
            pl.BlockSpec((3 * DIM, DIM), lambda i: (0, 0)),
            pl.BlockSpec((1, DIM), lambda i: (0, 0)),
        ],
        out_specs=[
            pl.BlockSpec((BM, DIM), lambda i: (i, 0)),
            pl.BlockSpec((1, 1), lambda i: (0, 0)),
        ],
        out_shape=[
            jax.ShapeDtypeStruct((N_NODES, DIM), jnp.float32),
            jax.ShapeDtypeStruct((1, 1), jnp.float32),
        ],
    )(partials, x, W, b.reshape(1, DIM))

    out = pl.pallas_call(
        _scale_body,
        grid=(grid,),
        in_specs=[
            pl.BlockSpec((BM, DIM), lambda i: (i, 0)),
            pl.BlockSpec((1, 1), lambda i: (0, 0)),
        ],
        out_specs=pl.BlockSpec((BM, DIM), lambda i: (i, 0)),
        out_shape=jax.ShapeDtypeStruct((N_NODES, DIM), jnp.float32),
    )(f, ssq)
    return out
